# Initial kernel scaffold; baseline (speedup 1.0000x reference)
import jax
import jax.numpy as jnp
from jax.experimental import pallas as pl


def kernel(x, rois):
    raise NotImplementedError("write your pallas kernel here")


# speedup vs baseline: 54.0367x; 54.0367x over previous
import jax
import jax.numpy as jnp
from jax import lax
from jax.experimental import pallas as pl
from jax.experimental.pallas import tpu as pltpu

N = 5000
NP = 5120
W = 160
NB = 32
BR = 512
THR = 0.5
NEG_INF = float("-inf")


def _scores_classes_cols(x4):
    s = jnp.max(x4, axis=1, keepdims=True)
    c0 = x4[:, 0:1] == s
    c1 = x4[:, 1:2] == s
    c2 = x4[:, 2:3] == s
    cls = jnp.where(c0, 0, jnp.where(c1, 1, jnp.where(c2, 2, 3)))
    return s, cls.astype(jnp.int32)


def _build_body(xj_ref, rj_ref, xi_ref, bi_ref, aw_ref):
    f32 = jnp.float32
    i32 = jnp.int32
    g = pl.program_id(0)

    x4 = xj_ref[:, :]
    sj, cls = _scores_classes_cols(x4)
    sj_eff = jnp.where(cls != 3, sj, NEG_INF)
    r4 = rj_ref[:, :]
    x1j, y1j = r4[:, 0:1], r4[:, 1:2]
    x2j, y2j = r4[:, 2:3], r4[:, 3:4]
    area_j = jnp.maximum(x2j - x1j, 0.0) * jnp.maximum(y2j - y1j, 0.0)
    idx_j = lax.broadcasted_iota(i32, (BR, 1), 0) + g * BR

    lane = lax.broadcasted_iota(i32, (1, W), 1)
    acc = jnp.zeros((BR, W), i32)
    for b in range(NB):
        xi = xi_ref[8 * b:8 * b + 8, :]
        si = jnp.max(xi[0:4, :], axis=0, keepdims=True)
        inv_i = ((xi[3:4, :] > xi[0:1, :])
                 & (xi[3:4, :] > xi[1:2, :])
                 & (xi[3:4, :] > xi[2:3, :]))
        si_eff = jnp.where(inv_i, NEG_INF, si)
        bi = bi_ref[8 * b:8 * b + 8, :]
        x1i, y1i = bi[0:1, :], bi[1:2, :]
        x2i, y2i = bi[2:3, :], bi[3:4, :]
        area_i = jnp.maximum(x2i - x1i, 0.0) * jnp.maximum(y2i - y1i, 0.0)
        idx_i = lane + W * b

        iw = jnp.maximum(jnp.minimum(x2j, x2i) - jnp.maximum(x1j, x1i), 0.0)
        ih = jnp.maximum(jnp.minimum(y2j, y2i) - jnp.maximum(y1j, y1i), 0.0)
        inter = iw * ih
        union = jnp.maximum(area_j + area_i - inter, 1e-9)
        conflict = inter > THR * union
        prec = (si_eff > sj_eff) | ((si_eff == sj_eff) & (idx_i < idx_j))
        bits = (conflict & prec).astype(i32)
        acc = acc | lax.shift_left(bits, b)
    aw_ref[:, :] = acc


def _solve_body(aw_ref, xp_ref, rp_ref, osc_ref, olab_ref, obox_ref):
    f32 = jnp.float32
    i32 = jnp.int32

    x4 = xp_ref[:, :]
    sj, cls = _scores_classes_cols(x4)
    valid = cls != 3
    r4 = rp_ref[:, :]

    ii = lax.broadcasted_iota(i32, (NP, W), 0)
    ww = lax.broadcasted_iota(i32, (NP, W), 1)
    bb = ii // W
    pw2 = lax.shift_left(jnp.ones((NP, W), i32), bb & 15).astype(f32)
    same = (ii % W) == ww
    plo = jnp.where(same & (bb < 16), pw2, 0.0)
    phi = jnp.where(same & (bb >= 16), pw2, 0.0)
    dn0 = (((0,), (0,)), ((), ()))

    def pack(keep_col):
        lo = lax.dot_general(keep_col, plo, dn0, preferred_element_type=f32)
        hi = lax.dot_general(keep_col, phi, dn0, preferred_element_type=f32)
        return lo.astype(i32) | lax.shift_left(hi.astype(i32), 16)

    def cond(c):
        return c[2]

    def body(c):
        keep, kw, _ = c
        nz = (aw_ref[:, :] & kw) != 0
        sup = jnp.max(nz.astype(f32), axis=1, keepdims=True)
        keep_new = 1.0 - sup
        changed = jnp.max(jnp.abs(keep_new - keep)) > 0.0
        return keep_new, pack(keep_new), changed

    keep0 = jnp.ones((NP, 1), f32)
    kw0 = jnp.full((1, W), -1, i32)
    keep_fin, _, _ = lax.while_loop(
        cond, body, (keep0, kw0, jnp.bool_(True)))

    keepv = keep_fin * valid.astype(f32)
    osc_ref[:, :] = sj * keepv
    olab_ref[:, :] = cls * keepv.astype(i32)
    obox_ref[:, :] = r4 * keepv


@jax.jit
def kernel(x, rois):
    f32 = jnp.float32
    xp = jnp.zeros((NP, 4), f32).at[:N, :].set(x)
    rp = jnp.zeros((NP, 4), f32).at[:N, :].set(rois)
    xi3 = jnp.zeros((NB, 8, W), f32).at[:, :4, :].set(
        xp.reshape(NB, W, 4).transpose(0, 2, 1)).reshape(NB * 8, W)
    bi3 = jnp.zeros((NB, 8, W), f32).at[:, :4, :].set(
        rp.reshape(NB, W, 4).transpose(0, 2, 1)).reshape(NB * 8, W)

    aw = pl.pallas_call(
        _build_body,
        grid=(NP // BR,),
        in_specs=[
            pl.BlockSpec((BR, 4), lambda g: (g, 0)),
            pl.BlockSpec((BR, 4), lambda g: (g, 0)),
            pl.BlockSpec((NB * 8, W), lambda g: (0, 0)),
            pl.BlockSpec((NB * 8, W), lambda g: (0, 0)),
        ],
        out_specs=pl.BlockSpec((BR, W), lambda g: (g, 0)),
        out_shape=jax.ShapeDtypeStruct((NP, W), jnp.int32),
    )(xp, rp, xi3, bi3)

    osc, olab, obox = pl.pallas_call(
        _solve_body,
        out_shape=[
            jax.ShapeDtypeStruct((NP, 1), f32),
            jax.ShapeDtypeStruct((NP, 1), jnp.int32),
            jax.ShapeDtypeStruct((NP, 4), f32),
        ],
    )(aw, xp, rp)
    return osc[:N, 0], olab[:N, 0], obox[:N, :]
